# baseline (speedup 1.0000x reference)
import functools

import jax
import jax.numpy as jnp
from jax import lax
from jax.experimental import pallas as pl
from jax.experimental.pallas import tpu as pltpu
from jax.experimental.pallas import tpu_sc as plsc

NC = 2
NS = 16
NW = NC * NS
CHUNK = 128
D = 128


def _mesh():
    return plsc.VectorSubcoreMesh(
        core_axis_name="c", subcore_axis_name="s", num_cores=NC, num_subcores=NS
    )


def _deg_call(dstr, zeros_rows, ones_rows, n_pad, k):
    rpt = n_pad // NS

    @functools.partial(
        pl.kernel,
        out_type=jax.ShapeDtypeStruct((NC * n_pad, D), jnp.float32),
        mesh=_mesh(),
        scratch_types=[
            pltpu.VMEM((k, CHUNK), jnp.int32),
            pltpu.VMEM((CHUNK, D), jnp.float32),
            pltpu.VMEM_SHARED((n_pad, D), jnp.float32),
        ],
    )
    def deg_k(dstr_hbm, z_hbm, ones_hbm, out_hbm, dst_v, ones_v, acc):
        c = lax.axis_index("c")
        s = lax.axis_index("s")
        w = c * NS + s
        pltpu.sync_copy(dstr_hbm.at[w], dst_v)
        pltpu.sync_copy(ones_hbm, ones_v)
        pltpu.sync_copy(z_hbm, acc.at[pl.ds(s * rpt, rpt)])
        plsc.subcore_barrier()

        def body(j, carry):
            pltpu.sync_copy(ones_v, acc.at[dst_v.at[j]], add=True)
            return carry

        lax.fori_loop(0, k, body, 0)
        plsc.subcore_barrier()
        pltpu.sync_copy(
            acc.at[pl.ds(s * rpt, rpt)],
            out_hbm.at[pl.ds(c * n_pad + s * rpt, rpt)],
        )

    return deg_k(dstr, zeros_rows, ones_rows)


def _scatter_call(hw2, srcr, dstr, zrows, n_pad, k):
    rpt = n_pad // NS

    @functools.partial(
        pl.kernel,
        out_type=jax.ShapeDtypeStruct((NC * n_pad, D), jnp.float32),
        mesh=_mesh(),
        scratch_types=[
            pltpu.VMEM((k, CHUNK), jnp.int32),
            pltpu.VMEM((k, CHUNK), jnp.int32),
            pltpu.VMEM((CHUNK, D), jnp.float32),
            pltpu.SemaphoreType.DMA,
            pltpu.VMEM_SHARED((n_pad, D), jnp.float32),
        ],
    )
    def sc_k(hw2_hbm, srcr_hbm, dstr_hbm, z_hbm, out_hbm,
             src_v, dst_v, rows_v, sem, acc):
        c = lax.axis_index("c")
        s = lax.axis_index("s")
        w = c * NS + s
        pltpu.sync_copy(srcr_hbm.at[w], src_v)
        pltpu.sync_copy(dstr_hbm.at[w], dst_v)
        pltpu.sync_copy(z_hbm, acc.at[pl.ds(s * rpt, rpt)])
        plsc.subcore_barrier()

        def body(j, carry):
            pltpu.async_copy(hw2_hbm.at[src_v.at[j]], rows_v, sem).wait()
            pltpu.sync_copy(rows_v, acc.at[dst_v.at[j]], add=True)
            return carry

        lax.fori_loop(0, k, body, 0)
        plsc.subcore_barrier()
        pltpu.sync_copy(
            acc.at[pl.ds(s * rpt, rpt)],
            out_hbm.at[pl.ds(c * n_pad + s * rpt, rpt)],
        )

    return sc_k(hw2, srcr, dstr, zrows)


def _row_block(n):
    for r in (1024, 800, 640, 512, 400, 320, 256, 200, 160, 128, 80, 40, 16, 8):
        if n % r == 0:
            return r
    return n


def _dis_call(d0, d1):
    n = d0.shape[0]
    r = _row_block(n)

    def k(a_ref, b_ref, o_ref):
        o_ref[...] = lax.rsqrt(a_ref[:, 0:1] + b_ref[:, 0:1] + 1.0)

    mat = pl.BlockSpec((r, D), lambda i: (i, 0))
    return pl.pallas_call(
        k,
        grid=(n // r,),
        in_specs=[mat, mat],
        out_specs=pl.BlockSpec((r, 1), lambda i: (i, 0)),
        out_shape=jax.ShapeDtypeStruct((n, 1), jnp.float32),
    )(d0, d1)


def _matmul_scale_call(h, w_mat, dis):
    n = h.shape[0]
    r = _row_block(n)

    def k(h_ref, w_ref, dis_ref, o_ref):
        o_ref[...] = (
            jnp.dot(
                h_ref[...],
                w_ref[...],
                preferred_element_type=jnp.float32,
                precision=lax.Precision.HIGHEST,
            )
            * dis_ref[...]
        )

    return pl.pallas_call(
        k,
        grid=(n // r,),
        in_specs=[
            pl.BlockSpec((r, D), lambda i: (i, 0)),
            pl.BlockSpec((D, D), lambda i: (0, 0)),
            pl.BlockSpec((r, 1), lambda i: (i, 0)),
        ],
        out_specs=pl.BlockSpec((r, D), lambda i: (i, 0)),
        out_shape=jax.ShapeDtypeStruct((n, D), jnp.float32),
    )(h, w_mat, dis)


def _post_call(p0, p1, hw2, dis, b, g, be, h):
    n = h.shape[0]
    r = _row_block(n)

    def k(p0_ref, p1_ref, hw_ref, dis_ref, b_ref, g_ref, be_ref, h_ref, o_ref):
        agg = dis_ref[...] * (p0_ref[...] + p1_ref[...] + hw_ref[...]) + b_ref[...]
        mu = jnp.mean(agg, axis=-1, keepdims=True)
        xc = agg - mu
        var = jnp.mean(xc * xc, axis=-1, keepdims=True)
        y = xc * lax.rsqrt(var + 1e-5) * g_ref[...] + be_ref[...]
        o_ref[...] = h_ref[...] + jnp.maximum(y, 0.0)

    mat = lambda: pl.BlockSpec((r, D), lambda i: (i, 0))
    vec = lambda: pl.BlockSpec((1, D), lambda i: (0, 0))
    return pl.pallas_call(
        k,
        grid=(n // r,),
        in_specs=[
            mat(),
            mat(),
            mat(),
            pl.BlockSpec((r, 1), lambda i: (i, 0)),
            vec(),
            vec(),
            vec(),
            mat(),
        ],
        out_specs=mat(),
        out_shape=jax.ShapeDtypeStruct((n, D), jnp.float32),
    )(p0, p1, hw2, dis, b, g, be, h)


def kernel(x, edge_index, W0, b0, g0, be0, W1, b1, g1, be1, W2, b2, g2, be2):
    n, d = x.shape
    e = edge_index.shape[1]
    n_pad = ((n + 1) + NS * 8 - 1) // (NS * 8) * (NS * 8)
    per = NW * CHUNK
    k = (e + per - 1) // per
    k = (k + 3) // 4 * 4
    e_pad = k * per
    pad = e_pad - e

    src = edge_index[0]
    dst = edge_index[1]
    srcp = jnp.concatenate(
        [src, jnp.zeros((pad,), jnp.int32)]
    ).reshape(NW, k, CHUNK)
    dstp = jnp.concatenate(
        [dst, jnp.full((pad,), n, jnp.int32)]
    ).reshape(NW, k, CHUNK)

    rpt = n_pad // NS
    ones_rows = jnp.ones((CHUNK, D), jnp.float32)
    zrows = jnp.zeros((rpt, D), jnp.float32)

    degraw = _deg_call(dstp, zrows, ones_rows, n_pad, k)
    dis = _dis_call(degraw[0:n], degraw[n_pad:n_pad + n])

    h = x
    for (w_mat, b, g, be) in ((W0, b0, g0, be0), (W1, b1, g1, be1), (W2, b2, g2, be2)):
        hw2 = _matmul_scale_call(h, w_mat, dis)
        part = _scatter_call(hw2, srcp, dstp, zrows, n_pad, k)
        h = _post_call(
            part[0:n],
            part[n_pad:n_pad + n],
            hw2,
            dis,
            b.reshape(1, D),
            g.reshape(1, D),
            be.reshape(1, D),
            h,
        )
    return h


# speedup vs baseline: 1.0984x; 1.0984x over previous
import functools

import jax
import jax.numpy as jnp
from jax import lax
from jax.experimental import pallas as pl
from jax.experimental.pallas import tpu as pltpu
from jax.experimental.pallas import tpu_sc as plsc

NC = 2
NS = 16
NW = NC * NS
CHUNK = 128
D = 128


def _mesh():
    return plsc.VectorSubcoreMesh(
        core_axis_name="c", subcore_axis_name="s", num_cores=NC, num_subcores=NS
    )


def _deg_call(dstr, zeros_rows, ones_rows, n_pad, k):
    rpt = n_pad // NS

    @functools.partial(
        pl.kernel,
        out_type=jax.ShapeDtypeStruct((NC * n_pad, D), jnp.float32),
        mesh=_mesh(),
        scratch_types=[
            pltpu.VMEM((k, CHUNK), jnp.int32),
            pltpu.VMEM((CHUNK, D), jnp.float32),
            pltpu.VMEM_SHARED((n_pad, D), jnp.float32),
        ],
    )
    def deg_k(dstr_hbm, z_hbm, ones_hbm, out_hbm, dst_v, ones_v, acc):
        c = lax.axis_index("c")
        s = lax.axis_index("s")
        w = c * NS + s
        pltpu.sync_copy(dstr_hbm.at[w], dst_v)
        pltpu.sync_copy(ones_hbm, ones_v)
        pltpu.sync_copy(z_hbm, acc.at[pl.ds(s * rpt, rpt)])
        plsc.subcore_barrier()

        def body(j, carry):
            pltpu.sync_copy(ones_v, acc.at[dst_v.at[j]], add=True)
            return carry

        lax.fori_loop(0, k, body, 0)
        plsc.subcore_barrier()
        pltpu.sync_copy(
            acc.at[pl.ds(s * rpt, rpt)],
            out_hbm.at[pl.ds(c * n_pad + s * rpt, rpt)],
        )

    return deg_k(dstr, zeros_rows, ones_rows)


def _scatter_call(hw2, srcr, dstr, zrows, n_pad, k):
    rpt = n_pad // NS

    @functools.partial(
        pl.kernel,
        out_type=jax.ShapeDtypeStruct((NC * n_pad, D), jnp.float32),
        mesh=_mesh(),
        scratch_types=[
            pltpu.VMEM((k, CHUNK), jnp.int32),
            pltpu.VMEM((k, CHUNK), jnp.int32),
            pltpu.VMEM((CHUNK, D), jnp.float32),
            pltpu.SemaphoreType.DMA,
            pltpu.VMEM_SHARED((n_pad, D), jnp.float32),
        ],
    )
    def sc_k(hw2_hbm, srcr_hbm, dstr_hbm, z_hbm, out_hbm,
             src_v, dst_v, rows_v, sem, acc):
        c = lax.axis_index("c")
        s = lax.axis_index("s")
        w = c * NS + s
        pltpu.sync_copy(srcr_hbm.at[w], src_v)
        pltpu.sync_copy(dstr_hbm.at[w], dst_v)
        pltpu.sync_copy(z_hbm, acc.at[pl.ds(s * rpt, rpt)])
        plsc.subcore_barrier()

        def body(j, carry):
            pltpu.async_copy(hw2_hbm.at[src_v.at[j]], rows_v, sem).wait()
            pltpu.sync_copy(rows_v, acc.at[dst_v.at[j]], add=True)
            return carry

        lax.fori_loop(0, k, body, 0)
        plsc.subcore_barrier()
        pltpu.sync_copy(
            acc.at[pl.ds(s * rpt, rpt)],
            out_hbm.at[pl.ds(c * n_pad + s * rpt, rpt)],
        )

    return sc_k(hw2, srcr, dstr, zrows)


def _row_block(n):
    for r in (1024, 800, 640, 512, 400, 320, 256, 200, 160, 128, 80, 40, 16, 8):
        if n % r == 0:
            return r
    return n


def _dis_call(d0, d1):
    n = d0.shape[0]
    r = _row_block(n)

    def k(a_ref, b_ref, o_ref):
        o_ref[...] = lax.rsqrt(a_ref[:, 0:1] + b_ref[:, 0:1] + 1.0)

    mat = pl.BlockSpec((r, D), lambda i: (i, 0))
    return pl.pallas_call(
        k,
        grid=(n // r,),
        in_specs=[mat, mat],
        out_specs=pl.BlockSpec((r, 1), lambda i: (i, 0)),
        out_shape=jax.ShapeDtypeStruct((n, 1), jnp.float32),
    )(d0, d1)


def _matmul_scale_call(h, w_mat, dis):
    n = h.shape[0]
    r = _row_block(n)

    def k(h_ref, w_ref, dis_ref, o_ref):
        o_ref[...] = (
            jnp.dot(
                h_ref[...],
                w_ref[...],
                preferred_element_type=jnp.float32,
                precision=lax.Precision.HIGHEST,
            )
            * dis_ref[...]
        )

    return pl.pallas_call(
        k,
        grid=(n // r,),
        in_specs=[
            pl.BlockSpec((r, D), lambda i: (i, 0)),
            pl.BlockSpec((D, D), lambda i: (0, 0)),
            pl.BlockSpec((r, 1), lambda i: (i, 0)),
        ],
        out_specs=pl.BlockSpec((r, D), lambda i: (i, 0)),
        out_shape=jax.ShapeDtypeStruct((n, D), jnp.float32),
    )(h, w_mat, dis)


def _post_call(p0, p1, hw2, dis, b, g, be, h):
    n = h.shape[0]
    r = _row_block(n)

    def k(p0_ref, p1_ref, hw_ref, dis_ref, b_ref, g_ref, be_ref, h_ref, o_ref):
        agg = dis_ref[...] * (p0_ref[...] + p1_ref[...] + hw_ref[...]) + b_ref[...]
        mu = jnp.mean(agg, axis=-1, keepdims=True)
        xc = agg - mu
        var = jnp.mean(xc * xc, axis=-1, keepdims=True)
        y = xc * lax.rsqrt(var + 1e-5) * g_ref[...] + be_ref[...]
        o_ref[...] = h_ref[...] + jnp.maximum(y, 0.0)

    mat = lambda: pl.BlockSpec((r, D), lambda i: (i, 0))
    vec = lambda: pl.BlockSpec((1, D), lambda i: (0, 0))
    return pl.pallas_call(
        k,
        grid=(n // r,),
        in_specs=[
            mat(),
            mat(),
            mat(),
            pl.BlockSpec((r, 1), lambda i: (i, 0)),
            vec(),
            vec(),
            vec(),
            mat(),
        ],
        out_specs=mat(),
        out_shape=jax.ShapeDtypeStruct((n, D), jnp.float32),
    )(p0, p1, hw2, dis, b, g, be, h)


def kernel(x, edge_index, W0, b0, g0, be0, W1, b1, g1, be1, W2, b2, g2, be2):
    n, d = x.shape
    e = edge_index.shape[1]
    n_pad = ((n + 1) + NS * 8 - 1) // (NS * 8) * (NS * 8)
    per = NW * CHUNK
    k = (e + per - 1) // per
    k = (k + 3) // 4 * 4
    e_pad = k * per
    pad = e_pad - e

    src = edge_index[0]
    dst = edge_index[1]
    ar = jnp.arange(pad, dtype=jnp.int32)

    def _tiles(v):
        return v.reshape(k, NW, CHUNK).transpose(1, 0, 2)

    srcp = _tiles(jnp.concatenate([src, jnp.full((pad,), n, jnp.int32)]))
    dstp = _tiles(jnp.concatenate([dst, ar % n]))
    dstp_deg = _tiles(jnp.concatenate([dst, n + ar % (n_pad - n)]))

    rpt = n_pad // NS
    ones_rows = jnp.ones((CHUNK, D), jnp.float32)
    zrows = jnp.zeros((rpt, D), jnp.float32)

    degraw = _deg_call(dstp_deg, zrows, ones_rows, n_pad, k)
    dis = _dis_call(degraw[0:n], degraw[n_pad:n_pad + n])

    h = x
    for (w_mat, b, g, be) in ((W0, b0, g0, be0), (W1, b1, g1, be1), (W2, b2, g2, be2)):
        hw2 = _matmul_scale_call(h, w_mat, dis)
        hw2_ext = jnp.concatenate([hw2, jnp.zeros((8, D), jnp.float32)])
        part = _scatter_call(hw2_ext, srcp, dstp, zrows, n_pad, k)
        h = _post_call(
            part[0:n],
            part[n_pad:n_pad + n],
            hw2,
            dis,
            b.reshape(1, D),
            g.reshape(1, D),
            be.reshape(1, D),
            h,
        )
    return h


# speedup vs baseline: 1.4062x; 1.2803x over previous
import functools

import jax
import jax.numpy as jnp
from jax import lax
from jax.experimental import pallas as pl
from jax.experimental.pallas import tpu as pltpu
from jax.experimental.pallas import tpu_sc as plsc

NC = 2
NS = 16
NW = NC * NS
CHUNK = 128
D = 128


def _mesh():
    return plsc.VectorSubcoreMesh(
        core_axis_name="c", subcore_axis_name="s", num_cores=NC, num_subcores=NS
    )


def _deg_call(dstr, zeros_rows, ones_rows, n_pad, k):
    rpt = n_pad // NS

    @functools.partial(
        pl.kernel,
        out_type=jax.ShapeDtypeStruct((NC * n_pad, D), jnp.float32),
        mesh=_mesh(),
        scratch_types=[
            pltpu.VMEM((k, CHUNK), jnp.int32),
            pltpu.VMEM((CHUNK, D), jnp.float32),
            pltpu.VMEM_SHARED((n_pad, D), jnp.float32),
        ],
    )
    def deg_k(dstr_hbm, z_hbm, ones_hbm, out_hbm, dst_v, ones_v, acc):
        c = lax.axis_index("c")
        s = lax.axis_index("s")
        w = c * NS + s
        pltpu.sync_copy(dstr_hbm.at[w], dst_v)
        pltpu.sync_copy(ones_hbm, ones_v)
        pltpu.sync_copy(z_hbm, acc.at[pl.ds(s * rpt, rpt)])
        plsc.subcore_barrier()

        def body(j, carry):
            pltpu.sync_copy(ones_v, acc.at[dst_v.at[j]], add=True)
            return carry

        lax.fori_loop(0, k, body, 0)
        plsc.subcore_barrier()
        pltpu.sync_copy(
            acc.at[pl.ds(s * rpt, rpt)],
            out_hbm.at[pl.ds(c * n_pad + s * rpt, rpt)],
        )

    return deg_k(dstr, zeros_rows, ones_rows)


def _scatter_call(hw2, srcr, dstr, zrows, n_pad, k):
    rpt = n_pad // NS

    @functools.partial(
        pl.kernel,
        out_type=jax.ShapeDtypeStruct((NC * n_pad, D), jnp.float32),
        mesh=_mesh(),
        scratch_types=[
            pltpu.VMEM((k, CHUNK), jnp.int32),
            pltpu.VMEM((k, CHUNK), jnp.int32),
            pltpu.VMEM((CHUNK, D), jnp.float32),
            pltpu.SemaphoreType.DMA,
            pltpu.VMEM_SHARED((n_pad, D), jnp.float32),
        ],
    )
    def sc_k(hw2_hbm, srcr_hbm, dstr_hbm, z_hbm, out_hbm,
             src_v, dst_v, rows_v, sem, acc):
        c = lax.axis_index("c")
        s = lax.axis_index("s")
        w = c * NS + s
        pltpu.sync_copy(srcr_hbm.at[w], src_v)
        pltpu.sync_copy(dstr_hbm.at[w], dst_v)
        pltpu.sync_copy(z_hbm, acc.at[pl.ds(s * rpt, rpt)])
        plsc.subcore_barrier()

        def body(j, carry):
            pltpu.async_copy(hw2_hbm.at[src_v.at[j]], rows_v, sem).wait()
            pltpu.sync_copy(rows_v, acc.at[dst_v.at[j]], add=True)
            return carry

        lax.fori_loop(0, k, body, 0)
        plsc.subcore_barrier()
        pltpu.sync_copy(
            acc.at[pl.ds(s * rpt, rpt)],
            out_hbm.at[pl.ds(c * n_pad + s * rpt, rpt)],
        )

    return sc_k(hw2, srcr, dstr, zrows)


def _row_block(n):
    for r in (1024, 800, 640, 512, 400, 320, 256, 200, 160, 128, 80, 40, 16, 8):
        if n % r == 0:
            return r
    return n


def _dis_call(d0, d1):
    n = d0.shape[0]
    r = _row_block(n)

    def k(a_ref, b_ref, o_ref):
        o_ref[...] = lax.rsqrt(a_ref[:, 0:1] + b_ref[:, 0:1] + 1.0)

    mat = pl.BlockSpec((r, D), lambda i: (i, 0))
    return pl.pallas_call(
        k,
        grid=(n // r,),
        in_specs=[mat, mat],
        out_specs=pl.BlockSpec((r, 1), lambda i: (i, 0)),
        out_shape=jax.ShapeDtypeStruct((n, 1), jnp.float32),
    )(d0, d1)


def _matmul_scale_call(h, w_mat, dis):
    n = h.shape[0]
    r = _row_block(n)

    def k(h_ref, w_ref, dis_ref, o_ref):
        o_ref[...] = (
            jnp.dot(
                h_ref[...],
                w_ref[...],
                preferred_element_type=jnp.float32,
                precision=lax.Precision.HIGHEST,
            )
            * dis_ref[...]
        )

    return pl.pallas_call(
        k,
        grid=(n // r,),
        in_specs=[
            pl.BlockSpec((r, D), lambda i: (i, 0)),
            pl.BlockSpec((D, D), lambda i: (0, 0)),
            pl.BlockSpec((r, 1), lambda i: (i, 0)),
        ],
        out_specs=pl.BlockSpec((r, D), lambda i: (i, 0)),
        out_shape=jax.ShapeDtypeStruct((n, D), jnp.float32),
    )(h, w_mat, dis)


def _post_call(p0, p1, hw2, dis, b, g, be, h):
    n = h.shape[0]
    r = _row_block(n)

    def k(p0_ref, p1_ref, hw_ref, dis_ref, b_ref, g_ref, be_ref, h_ref, o_ref):
        agg = dis_ref[...] * (p0_ref[...] + p1_ref[...] + hw_ref[...]) + b_ref[...]
        mu = jnp.mean(agg, axis=-1, keepdims=True)
        xc = agg - mu
        var = jnp.mean(xc * xc, axis=-1, keepdims=True)
        y = xc * lax.rsqrt(var + 1e-5) * g_ref[...] + be_ref[...]
        o_ref[...] = h_ref[...] + jnp.maximum(y, 0.0)

    mat = lambda: pl.BlockSpec((r, D), lambda i: (i, 0))
    vec = lambda: pl.BlockSpec((1, D), lambda i: (0, 0))
    return pl.pallas_call(
        k,
        grid=(n // r,),
        in_specs=[
            mat(),
            mat(),
            mat(),
            pl.BlockSpec((r, 1), lambda i: (i, 0)),
            vec(),
            vec(),
            vec(),
            mat(),
        ],
        out_specs=mat(),
        out_shape=jax.ShapeDtypeStruct((n, D), jnp.float32),
    )(p0, p1, hw2, dis, b, g, be, h)


def kernel(x, edge_index, W0, b0, g0, be0, W1, b1, g1, be1, W2, b2, g2, be2):
    n, d = x.shape
    e = edge_index.shape[1]
    n_pad = ((n + 1) + NS * 8 - 1) // (NS * 8) * (NS * 8)
    per = NW * CHUNK
    k = (e + per - 1) // per
    e_pad = k * per
    pad = e_pad - e

    src = edge_index[0]
    dst = edge_index[1]
    srcp = jnp.concatenate(
        [src, jnp.zeros((pad,), jnp.int32)]
    ).reshape(NW, k, CHUNK)
    dstp = jnp.concatenate(
        [dst, jnp.full((pad,), n, jnp.int32)]
    ).reshape(NW, k, CHUNK)

    rpt = n_pad // NS
    ones_rows = jnp.ones((CHUNK, D), jnp.float32)
    zrows = jnp.zeros((rpt, D), jnp.float32)

    degraw = _deg_call(dstp, zrows, ones_rows, n_pad, k)
    dis = _dis_call(degraw[0:n], degraw[n_pad:n_pad + n])

    h = x
    for (w_mat, b, g, be) in ((W0, b0, g0, be0), (W1, b1, g1, be1), (W2, b2, g2, be2)):
        hw2 = _matmul_scale_call(h, w_mat, dis)
        part = _scatter_call(hw2, srcp, dstp, zrows, n_pad, k)
        h = _post_call(
            part[0:n],
            part[n_pad:n_pad + n],
            hw2,
            dis,
            b.reshape(1, D),
            g.reshape(1, D),
            be.reshape(1, D),
            h,
        )
    return h
